# baseline (speedup 1.0000x reference)
import functools

import jax
import jax.numpy as jnp
from jax import lax
from jax.experimental import pallas as pl
from jax.experimental.pallas import tpu as pltpu
from jax.experimental.pallas import tpu_sc as plsc

_B, _F, _D, _V = 16384, 129, 128, 100
_NC, _NS, _L = 2, 16, 16
_NW = _NC * _NS
_RPW = _B // _NW
_BLK = 128
_NBLK = _RPW // _BLK


def _body(x_hbm, table_hbm, out_hbm, slab, idx2, emb0, emb1,
          dsem, gsem0, gsem1, wsem0, wsem1):
    wid = lax.axis_index("s") * _NC + lax.axis_index("c")
    row0 = wid * _RPW
    lanes = lax.iota(jnp.int32, _L)

    dense = pltpu.async_copy(
        x_hbm.at[pl.ds(row0, _RPW), pl.ds(0, _D)],
        out_hbm.at[pl.ds(row0, _RPW), pl.ds(0, _D)], dsem)
    pltpu.sync_copy(x_hbm.at[pl.ds(row0, _RPW), pl.ds(_F - 1, 1)],
                    slab.at[:, pl.ds(0, 1)])

    embs, gsems, wsems = [emb0, emb1], [gsem0, gsem1], [wsem0, wsem1]
    gathers = [None] * _NBLK
    writes = [None] * _NBLK
    for k in range(_NBLK):
        b = k & 1
        if k >= 2:
            writes[k - 2].wait()
        for i in range(_BLK // _L):
            acc = jnp.zeros((_L,), jnp.float32)
            for j in range(_L):
                s = slab[k * _BLK + i * _L + j, pl.ds(0, _L)][0]
                acc = jnp.where(lanes == j, s, acc)
            idx2[k, pl.ds(i * _L, _L)] = acc.astype(jnp.int32)
        gathers[k] = pltpu.async_copy(table_hbm.at[idx2.at[k]],
                                      embs[b], gsems[b])
        if k >= 1:
            gathers[k - 1].wait()
            writes[k - 1] = pltpu.async_copy(
                embs[1 - b],
                out_hbm.at[pl.ds(row0 + (k - 1) * _BLK, _BLK),
                           pl.ds(_D, _D)],
                wsems[1 - b])
    gathers[_NBLK - 1].wait()
    writes[_NBLK - 1] = pltpu.async_copy(
        embs[(_NBLK - 1) & 1],
        out_hbm.at[pl.ds(row0 + (_NBLK - 1) * _BLK, _BLK), pl.ds(_D, _D)],
        wsems[(_NBLK - 1) & 1])
    writes[_NBLK - 2].wait()
    writes[_NBLK - 1].wait()
    dense.wait()


@jax.jit
def kernel(x, table):
    mesh = plsc.VectorSubcoreMesh(core_axis_name="c", subcore_axis_name="s")
    f = pl.kernel(
        _body,
        mesh=mesh,
        out_type=jax.ShapeDtypeStruct((_B, 2 * _D), jnp.float32),
        scratch_types=[
            pltpu.VMEM((_RPW, _L), jnp.float32),
            pltpu.VMEM((_NBLK, _BLK), jnp.int32),
            pltpu.VMEM((_BLK, _D), jnp.float32),
            pltpu.VMEM((_BLK, _D), jnp.float32),
            pltpu.SemaphoreType.DMA,
            pltpu.SemaphoreType.DMA,
            pltpu.SemaphoreType.DMA,
            pltpu.SemaphoreType.DMA,
            pltpu.SemaphoreType.DMA,
        ],
        compiler_params=pltpu.CompilerParams(use_tc_tiling_on_sc=False),
    )
    return f(x, table)


# speedup vs baseline: 3.4487x; 3.4487x over previous
import functools

import jax
import jax.numpy as jnp
from jax import lax
from jax.experimental import pallas as pl
from jax.experimental.pallas import tpu as pltpu
from jax.experimental.pallas import tpu_sc as plsc

_B, _F, _D, _V = 16384, 129, 128, 100
_NC, _NS, _L = 2, 16, 16
_NW = _NC * _NS
_RPW = _B // _NW
_BLK = 128
_NBLK = _RPW // _BLK


def _body(x_hbm, table_hbm, out_hbm, xb0, xb1, emb0, emb1, idx2,
          isem0, isem1, gsem0, gsem1, dsem0, dsem1, esem0, esem1):
    wid = lax.axis_index("s") * _NC + lax.axis_index("c")
    row0 = wid * _RPW
    lanes = lax.iota(jnp.int32, _L)
    xbs, embs = [xb0, xb1], [emb0, emb1]
    isems, gsems = [isem0, isem1], [gsem0, gsem1]
    dsems, esems = [dsem0, dsem1], [esem0, esem1]

    def in_copy(k):
        return pltpu.async_copy(
            x_hbm.at[pl.ds(row0 + k * _BLK, _BLK), :], xbs[k & 1],
            isems[k & 1])

    ins = [None] * _NBLK
    gathers = [None] * _NBLK
    dws = [None] * _NBLK
    ews = [None] * _NBLK
    ins[0] = in_copy(0)
    for k in range(_NBLK):
        b = k & 1
        ins[k].wait()
        for i in range(_BLK // _L):
            acc = jnp.zeros((_L,), jnp.float32)
            for j in range(_L):
                s = xbs[b][i * _L + j, pl.ds(_F - _L, _L)][_L - 1]
                acc = jnp.where(lanes == j, s, acc)
            idx2[k, pl.ds(i * _L, _L)] = acc.astype(jnp.int32)
        if k >= 2:
            ews[k - 2].wait()
        gathers[k] = pltpu.async_copy(table_hbm.at[idx2.at[k]],
                                      embs[b], gsems[b])
        dws[k] = pltpu.async_copy(
            xbs[b].at[:, pl.ds(0, _D)],
            out_hbm.at[pl.ds(row0 + k * _BLK, _BLK), pl.ds(0, _D)],
            dsems[b])
        if k + 1 < _NBLK:
            if k >= 1:
                dws[k - 1].wait()
            ins[k + 1] = in_copy(k + 1)
        gathers[k].wait()
        ews[k] = pltpu.async_copy(
            embs[b],
            out_hbm.at[pl.ds(row0 + k * _BLK, _BLK), pl.ds(_D, _D)],
            esems[b])
    dws[_NBLK - 2].wait()
    dws[_NBLK - 1].wait()
    ews[_NBLK - 2].wait()
    ews[_NBLK - 1].wait()


@jax.jit
def kernel(x, table):
    mesh = plsc.VectorSubcoreMesh(core_axis_name="c", subcore_axis_name="s")
    f = pl.kernel(
        _body,
        mesh=mesh,
        out_type=jax.ShapeDtypeStruct((_B, 2 * _D), jnp.float32),
        scratch_types=[
            pltpu.VMEM((_BLK, _F), jnp.float32),
            pltpu.VMEM((_BLK, _F), jnp.float32),
            pltpu.VMEM((_BLK, _D), jnp.float32),
            pltpu.VMEM((_BLK, _D), jnp.float32),
            pltpu.VMEM((_NBLK, _BLK), jnp.int32),
        ] + [pltpu.SemaphoreType.DMA] * 8,
        compiler_params=pltpu.CompilerParams(use_tc_tiling_on_sc=False),
    )
    return f(x, table)


# speedup vs baseline: 5.5352x; 1.6050x over previous
import functools

import jax
import jax.numpy as jnp
from jax import lax
from jax.experimental import pallas as pl
from jax.experimental.pallas import tpu as pltpu
from jax.experimental.pallas import tpu_sc as plsc

_B, _F, _D, _V = 16384, 129, 128, 100
_NC, _NS, _L = 2, 16, 16
_NW = _NC * _NS
_RPW = _B // _NW
_BLK = 128
_NBLK = _RPW // _BLK
_TCROWS = 1024


def _tc_body(x_ref, out_ref, idx_ref):
    out_ref[...] = x_ref[:, : _D]
    idx_ref[...] = x_ref[:, _F - 1].astype(jnp.int32)


def _sc_body(table_hbm, idx_hbm, o_hbm, iv0, iv1, emb0, emb1,
             isem0, isem1, gsem0, gsem1, wsem0, wsem1):
    wid = lax.axis_index("s") * _NC + lax.axis_index("c")
    row0 = wid * _RPW
    ivs, embs = [iv0, iv1], [emb0, emb1]
    isems, gsems, wsems = [isem0, isem1], [gsem0, gsem1], [wsem0, wsem1]

    def idx_copy(k):
        return pltpu.async_copy(
            idx_hbm.at[pl.ds(row0 + k * _BLK, _BLK)], ivs[k & 1],
            isems[k & 1])

    ins = [None] * _NBLK
    gathers = [None] * _NBLK
    writes = [None] * _NBLK
    ins[0] = idx_copy(0)
    for k in range(_NBLK):
        b = k & 1
        ins[k].wait()
        if k >= 2:
            writes[k - 2].wait()
        gathers[k] = pltpu.async_copy(table_hbm.at[ivs[b]], embs[b],
                                      gsems[b])
        if k + 1 < _NBLK:
            ins[k + 1] = idx_copy(k + 1)
        gathers[k].wait()
        writes[k] = pltpu.async_copy(
            embs[b],
            o_hbm.at[pl.ds(row0 + k * _BLK, _BLK), pl.ds(_D, _D)],
            wsems[b])
    writes[_NBLK - 2].wait()
    writes[_NBLK - 1].wait()


@jax.jit
def kernel(x, table):
    out_init, idx = pl.pallas_call(
        _tc_body,
        grid=(_B // _TCROWS,),
        in_specs=[pl.BlockSpec((_TCROWS, _F), lambda i: (i, 0))],
        out_specs=[pl.BlockSpec((_TCROWS, _D), lambda i: (i, 0)),
                   pl.BlockSpec((_TCROWS,), lambda i: (i,))],
        out_shape=[jax.ShapeDtypeStruct((_B, 2 * _D), jnp.float32),
                   jax.ShapeDtypeStruct((_B,), jnp.int32)],
    )(x)

    oref = jax.new_ref(out_init)
    mesh = plsc.VectorSubcoreMesh(core_axis_name="c", subcore_axis_name="s")
    f = pl.kernel(
        _sc_body,
        mesh=mesh,
        out_type=(),
        scratch_types=[
            pltpu.VMEM((_BLK,), jnp.int32),
            pltpu.VMEM((_BLK,), jnp.int32),
            pltpu.VMEM((_BLK, _D), jnp.float32),
            pltpu.VMEM((_BLK, _D), jnp.float32),
        ] + [pltpu.SemaphoreType.DMA] * 6,
    )
    f(table, idx, oref)
    return jax.freeze(oref)


# speedup vs baseline: 5.7799x; 1.0442x over previous
import functools

import jax
import jax.numpy as jnp
from jax import lax
from jax.experimental import pallas as pl
from jax.experimental.pallas import tpu as pltpu
from jax.experimental.pallas import tpu_sc as plsc

_B, _F, _D, _V = 16384, 129, 128, 100
_NC, _NS, _L = 2, 16, 16
_NW = _NC * _NS
_RPW = _B // _NW
_BLK = 128
_NBLK = _RPW // _BLK
_TCROWS = 1024


def _tc_body(xm_ref, out_ref):
    out_ref[...] = xm_ref[...]


def _sc_body(table_hbm, idx_hbm, o_hbm, iv0, iv1, emb0, emb1,
             isem0, isem1, gsem0, gsem1, wsem0, wsem1):
    wid = lax.axis_index("s") * _NC + lax.axis_index("c")
    row0 = wid * _RPW
    ivs, embs = [iv0, iv1], [emb0, emb1]
    isems, gsems, wsems = [isem0, isem1], [gsem0, gsem1], [wsem0, wsem1]

    def idx_copy(k):
        return pltpu.async_copy(
            idx_hbm.at[pl.ds(row0 + k * _BLK, _BLK)], ivs[k & 1],
            isems[k & 1])

    ins = [None] * _NBLK
    gathers = [None] * _NBLK
    writes = [None] * _NBLK
    ins[0] = idx_copy(0)
    for k in range(_NBLK):
        b = k & 1
        ins[k].wait()
        if k >= 2:
            writes[k - 2].wait()
        gathers[k] = pltpu.async_copy(table_hbm.at[ivs[b]], embs[b],
                                      gsems[b])
        if k + 1 < _NBLK:
            ins[k + 1] = idx_copy(k + 1)
        gathers[k].wait()
        writes[k] = pltpu.async_copy(
            embs[b],
            o_hbm.at[pl.ds(row0 + k * _BLK, _BLK), pl.ds(_D, _D)],
            wsems[b])
    writes[_NBLK - 2].wait()
    writes[_NBLK - 1].wait()


@jax.jit
def kernel(x, table):
    idx = x[:, _F - 1].astype(jnp.int32)
    out_init = pl.pallas_call(
        _tc_body,
        grid=(_B // _TCROWS,),
        in_specs=[pl.BlockSpec((_TCROWS, _D), lambda i: (i, 0))],
        out_specs=pl.BlockSpec((_TCROWS, _D), lambda i: (i, 0)),
        out_shape=jax.ShapeDtypeStruct((_B, 2 * _D), jnp.float32),
    )(x)

    oref = jax.new_ref(out_init)
    mesh = plsc.VectorSubcoreMesh(core_axis_name="c", subcore_axis_name="s")
    f = pl.kernel(
        _sc_body,
        mesh=mesh,
        out_type=(),
        scratch_types=[
            pltpu.VMEM((_BLK,), jnp.int32),
            pltpu.VMEM((_BLK,), jnp.int32),
            pltpu.VMEM((_BLK, _D), jnp.float32),
            pltpu.VMEM((_BLK, _D), jnp.float32),
        ] + [pltpu.SemaphoreType.DMA] * 6,
    )
    f(table, idx, oref)
    return jax.freeze(oref)
